# Initial kernel scaffold; baseline (speedup 1.0000x reference)
import jax
import jax.numpy as jnp
from jax.experimental import pallas as pl


def kernel(likelihood_position, likelihood_count, local_cellxregion_ix):
    raise NotImplementedError("write your pallas kernel here")


# speedup vs baseline: 2.0981x; 2.0981x over previous
import functools

import jax
import jax.numpy as jnp
from jax import lax
from jax.experimental import pallas as pl
from jax.experimental.pallas import tpu as pltpu
from jax.experimental.pallas import tpu_sc as plsc

_N_CELLS = 200
_N_REGIONS = 500
_N_FRAGMENTS = 6400000
_NUM_SEG = _N_CELLS * _N_REGIONS

_NW = 32
_PER_W = _N_FRAGMENTS // _NW
_CHUNK = 2000
_NCH = _PER_W // _CHUNK

_mesh = plsc.VectorSubcoreMesh(core_axis_name="c", subcore_axis_name="s")


@functools.partial(
    pl.kernel,
    out_type=jax.ShapeDtypeStruct((_NW, _NUM_SEG), jnp.float32),
    mesh=_mesh,
    compiler_params=pltpu.CompilerParams(needs_layout_passes=False),
    scratch_types=[
        pltpu.VMEM((_NUM_SEG,), jnp.float32),
        pltpu.VMEM((_CHUNK,), jnp.int32),
        pltpu.VMEM((_CHUNK,), jnp.float32),
    ],
)
def _sc_segsum(ix_hbm, pos_hbm, out_hbm, acc, ixbuf, valbuf):
    wid = lax.axis_index("s") * 2 + lax.axis_index("c")

    zero16 = jnp.zeros((16,), jnp.float32)

    def zbody(i, carry):
        acc[pl.ds(i * 16, 16)] = zero16
        return carry

    lax.fori_loop(0, _NUM_SEG // 16, zbody, 0)

    base = wid * _PER_W

    def cbody(c, carry):
        off = base + c * _CHUNK
        pltpu.sync_copy(ix_hbm.at[pl.ds(off, _CHUNK)], ixbuf)
        pltpu.sync_copy(pos_hbm.at[pl.ds(off, _CHUNK)], valbuf)

        def vbody(j, inner):
            iv = ixbuf[pl.ds(j * 16, 16)]
            vv = valbuf[pl.ds(j * 16, 16)]
            plsc.addupdate_scatter(acc, [iv], vv)
            return inner

        lax.fori_loop(0, _CHUNK // 16, vbody, 0)
        return carry

    lax.fori_loop(0, _NCH, cbody, 0)

    pltpu.sync_copy(acc, out_hbm.at[wid])


def _tc_combine(parts_ref, cnt_ref, out_ref):
    out_ref[...] = jnp.sum(parts_ref[...], axis=0) + cnt_ref[...]


def kernel(likelihood_position, likelihood_count, local_cellxregion_ix):
    ix = local_cellxregion_ix.astype(jnp.int32)
    parts = _sc_segsum(ix, likelihood_position)
    parts3 = parts.reshape(_NW, _N_CELLS, _N_REGIONS)
    cnt = likelihood_count.reshape(_N_CELLS, _N_REGIONS)
    out = pl.pallas_call(
        _tc_combine,
        out_shape=jax.ShapeDtypeStruct((_N_CELLS, _N_REGIONS), jnp.float32),
    )(parts3, cnt)
    return out


# speedup vs baseline: 4.4672x; 2.1292x over previous
import functools

import jax
import jax.numpy as jnp
from jax import lax
from jax.experimental import pallas as pl
from jax.experimental.pallas import tpu as pltpu
from jax.experimental.pallas import tpu_sc as plsc

_N_CELLS = 200
_N_REGIONS = 500
_N_FRAGMENTS = 6400000
_NUM_SEG = _N_CELLS * _N_REGIONS

_NW = 32
_PER_W = _N_FRAGMENTS // _NW
_CHUNK = 4000
_NCH = _PER_W // _CHUNK
_NV = _CHUNK // 16
_UNROLL = 5

_SENT = jnp.int32(0x7FFFFFF0)

_mesh = plsc.VectorSubcoreMesh(core_axis_name="c", subcore_axis_name="s")


@functools.partial(
    pl.kernel,
    out_type=jax.ShapeDtypeStruct((_NW, _NUM_SEG), jnp.float32),
    mesh=_mesh,
    compiler_params=pltpu.CompilerParams(needs_layout_passes=False),
    scratch_types=[
        pltpu.VMEM((_NUM_SEG,), jnp.float32),
        pltpu.VMEM((2 * (_CHUNK + 16),), jnp.int32),
        pltpu.VMEM((2 * _CHUNK,), jnp.float32),
        pltpu.SemaphoreType.DMA((2,)),
        pltpu.SemaphoreType.DMA((2,)),
    ],
)
def _sc_segsum(ix_hbm, pos_hbm, out_hbm, acc, ixb, valb, six, sval):
    wid = lax.axis_index("s") * 2 + lax.axis_index("c")
    base = wid * _PER_W

    zero16 = jnp.zeros((16,), jnp.float32)

    def zbody(i, carry):
        for k in range(8):
            acc[pl.ds(i * 128 + k * 16, 16)] = zero16
        return carry

    lax.fori_loop(0, _NUM_SEG // 128, zbody, 0)
    acc[pl.ds(_NUM_SEG - 32, 16)] = zero16
    acc[pl.ds(_NUM_SEG - 16, 16)] = zero16

    def start_chunk(c, slot, n_ix):
        off = base + c * _CHUNK
        ib = slot * (_CHUNK + 16)
        vb = slot * _CHUNK
        pltpu.async_copy(
            ix_hbm.at[pl.ds(off, n_ix)], ixb.at[pl.ds(ib, n_ix)],
            six.at[slot])
        pltpu.async_copy(
            pos_hbm.at[pl.ds(off, _CHUNK)], valb.at[pl.ds(vb, _CHUNK)],
            sval.at[slot])

    def wait_chunk(c, slot, n_ix):
        off = base + c * _CHUNK
        ib = slot * (_CHUNK + 16)
        vb = slot * _CHUNK
        pltpu.make_async_copy(
            ix_hbm.at[pl.ds(off, n_ix)], ixb.at[pl.ds(ib, n_ix)],
            six.at[slot]).wait()
        pltpu.make_async_copy(
            pos_hbm.at[pl.ds(off, _CHUNK)], valb.at[pl.ds(vb, _CHUNK)],
            sval.at[slot]).wait()

    def vreg_step(slot, j, s_carry):
        ib = slot * (_CHUNK + 16)
        vb = slot * _CHUNK
        idx = ixb[pl.ds(ib + j * 16, 16)]
        idxn = ixb[pl.ds(ib + j * 16 + 1, 16)]
        val = valb[pl.ds(vb + j * 16, 16)]
        c = plsc.cumsum(val) + s_carry
        last = idx != idxn
        plsc.addupdate_scatter(acc, [idx], c, mask=last)
        minus_ok = last & (idxn < _NUM_SEG)
        plsc.addupdate_scatter(acc, [idxn], -c, mask=minus_ok)
        return c[15]

    def process_chunk(slot, s_carry):
        def vbody(g, s_c):
            for u in range(_UNROLL):
                s_c = vreg_step(slot, g * _UNROLL + u, s_c)
            return s_c

        return lax.fori_loop(0, _NV // _UNROLL, vbody, s_carry)

    start_chunk(0, 0, _CHUNK + 8)

    def outer(c, s_carry):
        slot = lax.rem(c, 2)
        nslot = 1 - slot

        @pl.when(c + 1 < _NCH - 1)
        def _():
            start_chunk(c + 1, nslot, _CHUNK + 8)

        @pl.when(c + 1 == _NCH - 1)
        def _():
            start_chunk(c + 1, nslot, _CHUNK)

        wait_chunk(c, slot, _CHUNK + 8)
        return process_chunk(slot, s_carry)

    s_carry = lax.fori_loop(0, _NCH - 1, outer, jnp.float32(0.0))

    lslot = (_NCH - 1) % 2
    wait_chunk(_NCH - 1, lslot, _CHUNK)
    ixb[pl.ds(lslot * (_CHUNK + 16) + _CHUNK, 16)] = jnp.full(
        (16,), _SENT, jnp.int32)
    process_chunk(lslot, s_carry)

    pltpu.sync_copy(acc, out_hbm.at[wid])


def _tc_combine(parts_ref, cnt_ref, out_ref):
    out_ref[...] = jnp.sum(parts_ref[...], axis=0) + cnt_ref[...]


def kernel(likelihood_position, likelihood_count, local_cellxregion_ix):
    ix = local_cellxregion_ix.astype(jnp.int32)
    parts = _sc_segsum(ix, likelihood_position)
    parts3 = parts.reshape(_NW, _N_CELLS, _N_REGIONS)
    cnt = likelihood_count.reshape(_N_CELLS, _N_REGIONS)
    out = pl.pallas_call(
        _tc_combine,
        out_shape=jax.ShapeDtypeStruct((_N_CELLS, _N_REGIONS), jnp.float32),
    )(parts3, cnt)
    return out
